# baseline (speedup 1.0000x reference)
import functools

import jax
import jax.numpy as jnp
from jax import lax
from jax.experimental import pallas as pl
from jax.experimental.pallas import tpu as pltpu
from jax.experimental.pallas import tpu_sc as plsc

B_TOTAL = 4 * 2048
D = 1024
NC, NS = 2, 16
NW = NC * NS
B_PER_W = B_TOTAL // NW
CH = 32
NCH = B_PER_W // CH


def _make_gather():
    mesh = plsc.VectorSubcoreMesh(core_axis_name="c", subcore_axis_name="s")

    @functools.partial(
        pl.kernel,
        mesh=mesh,
        out_type=jax.ShapeDtypeStruct((B_TOTAL, D), jnp.float32),
        scratch_types=[
            pltpu.VMEM((B_PER_W,), jnp.int32),
            pltpu.VMEM((CH, D), jnp.float32),
            pltpu.VMEM((CH, D), jnp.float32),
            pltpu.SemaphoreType.DMA,
            pltpu.SemaphoreType.DMA,
        ],
    )
    def k(idx_hbm, table_hbm, out_hbm, idx_v, rows_a, rows_b, gsem, osem):
        wid = lax.axis_index("s") * NC + lax.axis_index("c")
        base = wid * B_PER_W
        pltpu.sync_copy(idx_hbm.at[pl.ds(base, B_PER_W)], idx_v)
        bufs = (rows_a, rows_b)

        def gather(c):
            return pltpu.async_copy(
                table_hbm.at[idx_v.at[pl.ds(c * CH, CH)]], bufs[c % 2], gsem
            )

        def put(c):
            return pltpu.async_copy(
                bufs[c % 2], out_hbm.at[pl.ds(base + c * CH, CH)], osem
            )

        g = gather(0)
        o_prev = None
        for c in range(NCH):
            g.wait()
            if o_prev is not None:
                o_prev.wait()
            if c + 1 < NCH:
                g = gather(c + 1)
            o_prev = put(c)
        o_prev.wait()

    return k


_gather = _make_gather()


@jax.jit
def kernel(src_pos, table):
    idx = src_pos.reshape(-1).astype(jnp.int32)
    out = _gather(idx, table)
    return out.reshape(src_pos.shape + (D,))


# speedup vs baseline: 1.0389x; 1.0389x over previous
import functools

import jax
import jax.numpy as jnp
from jax import lax
from jax.experimental import pallas as pl
from jax.experimental.pallas import tpu as pltpu
from jax.experimental.pallas import tpu_sc as plsc

B_TOTAL = 4 * 2048
D = 1024
NC, NS = 2, 16
NW = NC * NS
B_PER_W = B_TOTAL // NW
CHUNKS = (96, 96, 64)
CH_MAX = max(CHUNKS)


def _make_gather():
    mesh = plsc.VectorSubcoreMesh(core_axis_name="c", subcore_axis_name="s")

    @functools.partial(
        pl.kernel,
        mesh=mesh,
        out_type=jax.ShapeDtypeStruct((B_TOTAL, D), jnp.float32),
        scratch_types=[
            pltpu.VMEM((B_PER_W,), jnp.int32),
            pltpu.VMEM((CH_MAX, D), jnp.float32),
            pltpu.SemaphoreType.DMA,
        ],
    )
    def k(idx_hbm, table_hbm, out_hbm, idx_v, rows_v, sem):
        wid = lax.axis_index("s") * NC + lax.axis_index("c")
        base = wid * B_PER_W
        pltpu.sync_copy(idx_hbm.at[pl.ds(base, B_PER_W)], idx_v)
        off = 0
        for ch in CHUNKS:
            pltpu.async_copy(
                table_hbm.at[idx_v.at[pl.ds(off, ch)]],
                rows_v.at[pl.ds(0, ch)],
                sem,
            ).wait()
            pltpu.sync_copy(
                rows_v.at[pl.ds(0, ch)], out_hbm.at[pl.ds(base + off, ch)]
            )
            off += ch

    return k


_gather = _make_gather()


@jax.jit
def kernel(src_pos, table):
    idx = src_pos.reshape(-1).astype(jnp.int32)
    out = _gather(idx, table)
    return out.reshape(src_pos.shape + (D,))
